# baseline (device time: 21994 ns/iter reference)
import jax
import jax.numpy as jnp
from jax import lax
from jax.experimental import pallas as pl
from jax.experimental.pallas import tpu as pltpu

N_DEV = 4


def kernel(partial, resid, gamma):
    m, d = resid.shape
    x = partial.reshape(m, d)
    g = gamma.reshape(1, d)

    def body(x_ref, resid_ref, gamma_ref, out_ref, comm_ref, send_sems, recv_sems):
        i = lax.axis_index("i")
        px = N_DEV - 1 - i
        py = i + 1 - 2 * (i % 2)

        barrier_sem = pltpu.get_barrier_semaphore()
        for nbr in (px, py):
            pl.semaphore_signal(
                barrier_sem, inc=1,
                device_id=(nbr,), device_id_type=pl.DeviceIdType.MESH,
            )
        pl.semaphore_wait(barrier_sem, 2)

        acc = x_ref[:, :]
        comm_ref[0, :, :] = acc.astype(jnp.bfloat16)
        r1 = pltpu.make_async_remote_copy(
            src_ref=comm_ref.at[0],
            dst_ref=comm_ref.at[1],
            send_sem=send_sems.at[0],
            recv_sem=recv_sems.at[0],
            device_id=(px,),
            device_id_type=pl.DeviceIdType.MESH,
        )
        r1.start()
        r1.wait()
        acc = acc + comm_ref[1, :, :].astype(jnp.float32)

        comm_ref[2, :, :] = acc.astype(jnp.bfloat16)
        r2 = pltpu.make_async_remote_copy(
            src_ref=comm_ref.at[2],
            dst_ref=comm_ref.at[3],
            send_sem=send_sems.at[1],
            recv_sem=recv_sems.at[1],
            device_id=(py,),
            device_id_type=pl.DeviceIdType.MESH,
        )
        r2.start()
        r2.wait()
        acc = acc + comm_ref[3, :, :].astype(jnp.float32)

        y = acc + resid_ref[:, :]
        rms = jnp.sqrt(jnp.mean(y * y, axis=-1, keepdims=True) + 1e-6)
        out_ref[:, :] = y / rms * gamma_ref[:, :]

    return pl.pallas_call(
        body,
        out_shape=jax.ShapeDtypeStruct((m, d), jnp.float32),
        in_specs=[
            pl.BlockSpec(memory_space=pltpu.VMEM),
            pl.BlockSpec(memory_space=pltpu.VMEM),
            pl.BlockSpec(memory_space=pltpu.VMEM),
        ],
        out_specs=pl.BlockSpec(memory_space=pltpu.VMEM),
        scratch_shapes=[
            pltpu.VMEM((4, m, d), jnp.bfloat16),
            pltpu.SemaphoreType.DMA((2,)),
            pltpu.SemaphoreType.DMA((2,)),
        ],
        compiler_params=pltpu.CompilerParams(collective_id=0),
    )(x, resid, g)


# device time: 16415 ns/iter; 1.3399x vs baseline; 1.3399x over previous
import jax
import jax.numpy as jnp
from jax import lax
from jax.experimental import pallas as pl
from jax.experimental.pallas import tpu as pltpu

N_DEV = 4

S_SEND_A1, S_SEND_B1, S_RECV_A1, S_RECV_B1 = 0, 1, 2, 3
S_SEND_A2, S_SEND_B2, S_RECV_A2, S_RECV_B2 = 4, 5, 6, 7


def kernel(partial, resid, gamma):
    m, d = resid.shape
    h = m // 2
    x = partial.reshape(m, d)
    g = gamma.reshape(1, d)

    def body(x_ref, resid_ref, gamma_ref, out_ref, comm_ref, send_sems, recv_sems):
        i = lax.axis_index("i")
        px = N_DEV - 1 - i
        py = i + 1 - 2 * (i % 2)

        def exchange(send_slot, recv_slot, sem, dev):
            return pltpu.make_async_remote_copy(
                src_ref=comm_ref.at[send_slot],
                dst_ref=comm_ref.at[recv_slot],
                send_sem=send_sems.at[sem],
                recv_sem=recv_sems.at[sem],
                device_id=(dev,),
                device_id_type=pl.DeviceIdType.MESH,
            )

        barrier_sem = pltpu.get_barrier_semaphore()
        for nbr in (px, py):
            pl.semaphore_signal(
                barrier_sem, inc=1,
                device_id=(nbr,), device_id_type=pl.DeviceIdType.MESH,
            )
        pl.semaphore_wait(barrier_sem, 2)

        xa = x_ref[0:h, :]
        xb = x_ref[h : 2 * h, :]
        comm_ref[S_SEND_A1, :, :] = xa.astype(jnp.bfloat16)
        comm_ref[S_SEND_B1, :, :] = xb.astype(jnp.bfloat16)
        r1a = exchange(S_SEND_A1, S_RECV_A1, 0, px)
        r1b = exchange(S_SEND_B1, S_RECV_B1, 1, py)
        r1a.start()
        r1b.start()

        r1a.wait()
        acc_a = xa + comm_ref[S_RECV_A1, :, :].astype(jnp.float32)
        comm_ref[S_SEND_A2, :, :] = acc_a.astype(jnp.bfloat16)
        r2a = exchange(S_SEND_A2, S_RECV_A2, 2, py)
        r2a.start()

        r1b.wait()
        acc_b = xb + comm_ref[S_RECV_B1, :, :].astype(jnp.float32)
        comm_ref[S_SEND_B2, :, :] = acc_b.astype(jnp.bfloat16)
        r2b = exchange(S_SEND_B2, S_RECV_B2, 3, px)
        r2b.start()

        gam = gamma_ref[:, :]
        r2a.wait()
        ya = acc_a + comm_ref[S_RECV_A2, :, :].astype(jnp.float32) + resid_ref[0:h, :]
        rms_a = jnp.sqrt(jnp.mean(ya * ya, axis=-1, keepdims=True) + 1e-6)
        out_ref[0:h, :] = ya / rms_a * gam

        r2b.wait()
        yb = (
            acc_b
            + comm_ref[S_RECV_B2, :, :].astype(jnp.float32)
            + resid_ref[h : 2 * h, :]
        )
        rms_b = jnp.sqrt(jnp.mean(yb * yb, axis=-1, keepdims=True) + 1e-6)
        out_ref[h : 2 * h, :] = yb / rms_b * gam

    return pl.pallas_call(
        body,
        out_shape=jax.ShapeDtypeStruct((m, d), jnp.float32),
        in_specs=[
            pl.BlockSpec(memory_space=pltpu.VMEM),
            pl.BlockSpec(memory_space=pltpu.VMEM),
            pl.BlockSpec(memory_space=pltpu.VMEM),
        ],
        out_specs=pl.BlockSpec(memory_space=pltpu.VMEM),
        scratch_shapes=[
            pltpu.VMEM((8, h, d), jnp.bfloat16),
            pltpu.SemaphoreType.DMA((4,)),
            pltpu.SemaphoreType.DMA((4,)),
        ],
        compiler_params=pltpu.CompilerParams(collective_id=0),
    )(x, resid, g)
